# baseline (speedup 1.0000x reference)
import functools

import jax
import jax.numpy as jnp
from jax import lax
from jax.experimental import pallas as pl
from jax.experimental.pallas import tpu as pltpu
from jax.experimental.pallas import tpu_sc as plsc

NC = 2
NS = 16
NW = NC * NS
L = 16
CHUNK = 64
NBUF = 2


def _body(num_rel, n_per_worker, ent_hbm, rel_hbm, h_hbm, t_hbm,
          r_hbm, out_hbm, hidx, tidx, ridx, rel, cidx, tidxb, rows, score,
          gsems, tsems, ssems):
    wid = lax.axis_index("s") * NC + lax.axis_index("c")
    base = wid * n_per_worker
    pltpu.sync_copy(h_hbm.at[pl.ds(base, n_per_worker)], hidx)
    pltpu.sync_copy(t_hbm.at[pl.ds(base, n_per_worker)], tidx)
    pltpu.sync_copy(r_hbm.at[pl.ds(base, n_per_worker)], ridx)
    pltpu.sync_copy(rel_hbm, rel)

    iota = lax.iota(jnp.int32, L)
    n_chunks = n_per_worker // CHUNK

    def prep_fire(k, b):
        off = k * CHUNK
        first = plsc.load_gather(hidx, [jnp.full((L,), off, jnp.int32)])
        hv, tv, rv, m = [], [], [], None
        for j in range(4):
            sl = pl.ds(off + j * L, L)
            hv.append(hidx[sl])
            tv.append(tidx[sl])
            rv.append(ridx[sl])
            e = hv[j] == first
            m = e if m is None else (m & e)
        is_neg = plsc.all_reduce_population_count(m) == L
        for j in range(4):
            nh = jnp.where(is_neg, hv[j], tv[j])
            nt = jnp.where(is_neg, tv[j], hv[j])
            cidx[b][pl.ds(j * L, L)] = nh + nh
            cidx[b][pl.ds(CHUNK + j * L, L)] = nh + nh + 1
            tidxb[b][pl.ds(j * L, L)] = nt + nt
            tidxb[b][pl.ds(CHUNK + j * L, L)] = nt + nt + 1
            ridx[pl.ds(off + j * L, L)] = jnp.where(is_neg, rv[j],
                                                    rv[j] + num_rel)
        pltpu.async_copy(ent_hbm.at[cidx[b]], rows[b].at[pl.ds(0, 128)],
                         gsems[b])
        pltpu.async_copy(ent_hbm.at[tidxb[b]], rows[b].at[pl.ds(128, 128)],
                         tsems[b])

    def wait_gather(b):
        pltpu.make_async_copy(ent_hbm.at[cidx[b]], rows[b].at[pl.ds(0, 128)],
                              gsems[b]).wait()
        pltpu.make_async_copy(ent_hbm.at[tidxb[b]],
                              rows[b].at[pl.ds(128, 128)], tsems[b]).wait()

    def wait_score(b):
        pltpu.make_async_copy(score[b], out_hbm.at[pl.ds(0, CHUNK)],
                              ssems[b]).wait()

    def compute(k, b):
        off = k * CHUNK
        zi = jnp.zeros((L,), jnp.int32)
        dim = 64
        for g in range(CHUNK // L):
            ev = iota + g * L
            nrv = ridx[pl.ds(off + g * L, L)]
            hbase = ev * 32
            rbase = nrv * dim

            def dbody(i, carry):
                ob, a0, a1, a2, a3 = carry
                accs = [a0, a1, a2, a3]
                for u in range(16):
                    offv = (ob + u) & (dim - 1)
                    adj = jnp.where(offv >= 32, 2016, 0)
                    fh = hbase + offv + adj
                    hvv = plsc.load_gather(rows[b], [zi, fh])
                    tvv = plsc.load_gather(rows[b], [zi, fh + 4096])
                    rvv = plsc.load_gather(rel, [zi, rbase + offv])
                    accs[u % 4] = accs[u % 4] + hvv * tvv * rvv
                return (ob + 16, *accs)

            z = jnp.zeros((L,), jnp.float32)
            res = lax.fori_loop(0, 4, dbody, (iota, z, z, z, z))
            score[b][pl.ds(g * L, L)] = ((res[1] + res[2])
                                         + (res[3] + res[4]))
        pltpu.async_copy(score[b], out_hbm.at[pl.ds(base + off, CHUNK)],
                         ssems[b])

    for b in range(NBUF - 1):
        prep_fire(b, b)

    def outer(c, _):
        k0 = c * NBUF
        for b in range(NBUF):
            k = k0 + b
            nb = (b + NBUF - 1) % NBUF

            @pl.when(k + NBUF - 1 < n_chunks)
            def _():
                prep_fire(k + NBUF - 1, nb)

            wait_gather(b)

            @pl.when(c > 0)
            def _():
                wait_score(b)

            compute(k, b)
        return 0

    lax.fori_loop(0, n_chunks // NBUF, outer, 0)
    for b in range(NBUF):
        wait_score(b)


def kernel(entity_emb, relation_emb, h_index, t_index, r_index):
    shape = h_index.shape
    total = h_index.size
    num_rel = relation_emb.shape[0] // 2
    dim = entity_emb.shape[1]
    n_per_worker = total // NW

    mesh = plsc.VectorSubcoreMesh(core_axis_name="c", subcore_axis_name="s")
    body = functools.partial(_body, num_rel, n_per_worker)
    run = pl.kernel(
        body,
        out_type=jax.ShapeDtypeStruct((total,), jnp.float32),
        mesh=mesh,
        compiler_params=pltpu.CompilerParams(
            needs_layout_passes=False, use_tc_tiling_on_sc=False,
            disable_bounds_checks=True, disable_semaphore_checks=True),
        scratch_types=[
            pltpu.VMEM((n_per_worker,), jnp.int32),
            pltpu.VMEM((n_per_worker,), jnp.int32),
            pltpu.VMEM((n_per_worker,), jnp.int32),
            pltpu.VMEM((2 * num_rel, dim), jnp.float32),
            [pltpu.VMEM((2 * CHUNK,), jnp.int32)] * NBUF,
            [pltpu.VMEM((2 * CHUNK,), jnp.int32)] * NBUF,
            [pltpu.VMEM((4 * CHUNK, 32), jnp.float32)] * NBUF,
            [pltpu.VMEM((CHUNK,), jnp.float32)] * NBUF,
            [pltpu.SemaphoreType.DMA] * NBUF,
            [pltpu.SemaphoreType.DMA] * NBUF,
            [pltpu.SemaphoreType.DMA] * NBUF,
        ],
    )
    ent2 = entity_emb.reshape(2 * entity_emb.shape[0], 32)
    out = run(ent2, relation_emb, h_index.reshape(-1),
              t_index.reshape(-1), r_index.reshape(-1))
    return out.reshape(shape)


# speedup vs baseline: 1.1011x; 1.1011x over previous
import functools

import jax
import jax.numpy as jnp
from jax import lax
from jax.experimental import pallas as pl
from jax.experimental.pallas import tpu as pltpu
from jax.experimental.pallas import tpu_sc as plsc

NC = 2
NS = 16
NW = NC * NS
L = 16
CHUNK = 64
NBUF = 2


def _body(num_rel, n_per_worker, ent_hbm, rel_hbm, h_hbm, t_hbm,
          r_hbm, out_hbm, hidx, tidx, ridx, rel, cidx, rows, score,
          gsems, ssems):
    wid = lax.axis_index("s") * NC + lax.axis_index("c")
    base = wid * n_per_worker
    pltpu.sync_copy(h_hbm.at[pl.ds(base, n_per_worker)], hidx)
    pltpu.sync_copy(t_hbm.at[pl.ds(base, n_per_worker)], tidx)
    pltpu.sync_copy(r_hbm.at[pl.ds(base, n_per_worker)], ridx)
    pltpu.sync_copy(rel_hbm, rel)

    iota = lax.iota(jnp.int32, L)
    n_chunks = n_per_worker // CHUNK

    def prep_fire(k, b):
        off = k * CHUNK
        first = plsc.load_gather(hidx, [jnp.full((L,), off, jnp.int32)])
        hv, tv, rv, m = [], [], [], None
        for j in range(4):
            sl = pl.ds(off + j * L, L)
            hv.append(hidx[sl])
            tv.append(tidx[sl])
            rv.append(ridx[sl])
            e = hv[j] == first
            m = e if m is None else (m & e)
        is_neg = plsc.all_reduce_population_count(m) == L
        for j in range(4):
            nh = jnp.where(is_neg, hv[j], tv[j])
            nt = jnp.where(is_neg, tv[j], hv[j])
            cidx[b][pl.ds(j * L, L)] = nh + nh
            cidx[b][pl.ds(CHUNK + j * L, L)] = nt + nt
            ridx[pl.ds(off + j * L, L)] = jnp.where(is_neg, rv[j],
                                                    rv[j] + num_rel)
        pltpu.async_copy(ent_hbm.at[cidx[b]], rows[b], gsems[b])

    def wait_gather(b):
        pltpu.make_async_copy(ent_hbm.at[cidx[b]], rows[b], gsems[b]).wait()

    def wait_score(b):
        pltpu.make_async_copy(score[b], out_hbm.at[pl.ds(0, CHUNK)],
                              ssems[b]).wait()

    def compute(k, b):
        off = k * CHUNK
        zi = jnp.zeros((L,), jnp.int32)
        dim = 64
        for g in range(CHUNK // L):
            ev = iota + g * L
            nrv = ridx[pl.ds(off + g * L, L)]
            hbase = ev * 64
            tbase = hbase + CHUNK * 64
            rbase = nrv * dim

            def dbody(i, carry):
                ob, a0, a1, a2, a3 = carry
                accs = [a0, a1, a2, a3]
                for u in range(16):
                    offv = (ob + u) & (dim - 1)
                    hvv = plsc.load_gather(rows[b], [zi, hbase + offv])
                    tvv = plsc.load_gather(rows[b], [zi, tbase + offv])
                    rvv = plsc.load_gather(rel, [zi, rbase + offv])
                    accs[u % 4] = accs[u % 4] + hvv * tvv * rvv
                return (ob + 16, *accs)

            z = jnp.zeros((L,), jnp.float32)
            res = lax.fori_loop(0, 4, dbody, (iota, z, z, z, z))
            score[b][pl.ds(g * L, L)] = ((res[1] + res[2])
                                         + (res[3] + res[4]))
        pltpu.async_copy(score[b], out_hbm.at[pl.ds(base + off, CHUNK)],
                         ssems[b])

    for b in range(NBUF - 1):
        prep_fire(b, b)

    def outer(c, _):
        k0 = c * NBUF
        for b in range(NBUF):
            k = k0 + b
            nb = (b + NBUF - 1) % NBUF

            @pl.when(k + NBUF - 1 < n_chunks)
            def _():
                prep_fire(k + NBUF - 1, nb)

            wait_gather(b)

            @pl.when(c > 0)
            def _():
                wait_score(b)

            compute(k, b)
        return 0

    lax.fori_loop(0, n_chunks // NBUF, outer, 0)
    for b in range(NBUF):
        wait_score(b)


def kernel(entity_emb, relation_emb, h_index, t_index, r_index):
    shape = h_index.shape
    total = h_index.size
    num_rel = relation_emb.shape[0] // 2
    dim = entity_emb.shape[1]
    n_per_worker = total // NW

    mesh = plsc.VectorSubcoreMesh(core_axis_name="c", subcore_axis_name="s")
    body = functools.partial(_body, num_rel, n_per_worker)
    run = pl.kernel(
        body,
        out_type=jax.ShapeDtypeStruct((total,), jnp.float32),
        mesh=mesh,
        compiler_params=pltpu.CompilerParams(
            needs_layout_passes=False, use_tc_tiling_on_sc=False,
            disable_bounds_checks=True, disable_semaphore_checks=True),
        scratch_types=[
            pltpu.VMEM((n_per_worker,), jnp.int32),
            pltpu.VMEM((n_per_worker,), jnp.int32),
            pltpu.VMEM((n_per_worker,), jnp.int32),
            pltpu.VMEM((2 * num_rel, dim), jnp.float32),
            [pltpu.VMEM((2 * CHUNK,), jnp.int32)] * NBUF,
            [pltpu.VMEM((2 * CHUNK, 64), jnp.float32)] * NBUF,
            [pltpu.VMEM((CHUNK,), jnp.float32)] * NBUF,
            [pltpu.SemaphoreType.DMA] * NBUF,
            [pltpu.SemaphoreType.DMA] * NBUF,
        ],
    )
    ent2 = jnp.pad(entity_emb, ((0, 0), (0, 64))).reshape(
        2 * entity_emb.shape[0], 64)
    out = run(ent2, relation_emb, h_index.reshape(-1),
              t_index.reshape(-1), r_index.reshape(-1))
    return out.reshape(shape)
